# Initial kernel scaffold; baseline (speedup 1.0000x reference)
import jax
import jax.numpy as jnp
from jax.experimental import pallas as pl


def kernel(x, edge_index, climber_features, W1, b1, W2, b2, Wg, bg, Wc1, bc1, Wc2, bc2):
    raise NotImplementedError("write your pallas kernel here")


# speedup vs baseline: 2.6998x; 2.6998x over previous
import functools

import jax
import jax.numpy as jnp
from jax.experimental import pallas as pl
from jax.experimental.pallas import tpu as pltpu

N = 10000
E = 320000
D_IN = 128
HID = 256
RB = 1000


def _scale_mm_kernel(dinv_ref, x_ref, w_ref, o_ref):
    xw = jnp.dot(x_ref[...], w_ref[...], preferred_element_type=jnp.float32)
    o_ref[...] = dinv_ref[...] * xw


def _mid_kernel(s_ref, y_ref, dinv_ref, b_ref, w_ref, o_ref):
    h = jnp.maximum(dinv_ref[...] * (s_ref[...] + y_ref[...]) + b_ref[...], 0.0)
    o_ref[...] = dinv_ref[...] * jnp.dot(h, w_ref[...],
                                         preferred_element_type=jnp.float32)


def _pool_kernel(s_ref, y_ref, dinv_ref, b_ref, acc_ref):
    @pl.when(pl.program_id(0) == 0)
    def _():
        acc_ref[...] = jnp.zeros_like(acc_ref)

    h = jnp.maximum(dinv_ref[...] * (s_ref[...] + y_ref[...]) + b_ref[...], 0.0)
    acc_ref[...] += jnp.sum(h, axis=0, keepdims=True)


def _head_kernel(ps_ref, cf_ref, wg_ref, bg_ref, wc1_ref, bc1_ref,
                 wc2_ref, bc2_ref, o_ref):
    pooled = ps_ref[...] * (1.0 / N)
    g = jnp.maximum(jnp.dot(cf_ref[...], wg_ref[...],
                            preferred_element_type=jnp.float32)
                    + bg_ref[...], 0.0)
    out = jnp.concatenate([pooled, g], axis=1)
    h = jnp.maximum(jnp.dot(out, wc1_ref[...],
                            preferred_element_type=jnp.float32)
                    + bc1_ref[...], 0.0)
    o_ref[...] = jnp.dot(h, wc2_ref[...],
                         preferred_element_type=jnp.float32) + bc2_ref[...]


def kernel(x, edge_index, climber_features, W1, b1, W2, b2, Wg, bg,
           Wc1, bc1, Wc2, bc2):
    src = edge_index[0]
    dst = edge_index[1]

    deg = jax.ops.segment_sum(jnp.ones((E,), jnp.float32), dst,
                              num_segments=N) + 1.0
    dinv = jax.lax.rsqrt(jnp.maximum(deg, 1.0))[:, None]

    grid = (N // RB,)
    row_spec = lambda d: pl.BlockSpec((RB, d), lambda i: (i, 0))
    dinv_spec = pl.BlockSpec((RB, 1), lambda i: (i, 0))

    def scale_mm(dinv2, xin, W):
        d = W.shape[1]
        return pl.pallas_call(
            _scale_mm_kernel,
            grid=grid,
            in_specs=[dinv_spec, row_spec(xin.shape[1]),
                      pl.BlockSpec((W.shape[0], d), lambda i: (0, 0))],
            out_specs=row_spec(d),
            out_shape=jax.ShapeDtypeStruct((N, d), jnp.float32),
        )(dinv2, xin, W)

    y1 = scale_mm(dinv, x, W1)
    s1 = jax.ops.segment_sum(y1[src], dst, num_segments=N)

    y2 = pl.pallas_call(
        _mid_kernel,
        grid=grid,
        in_specs=[row_spec(HID), row_spec(HID), dinv_spec,
                  pl.BlockSpec((1, HID), lambda i: (0, 0)),
                  pl.BlockSpec((HID, HID), lambda i: (0, 0))],
        out_specs=row_spec(HID),
        out_shape=jax.ShapeDtypeStruct((N, HID), jnp.float32),
    )(s1, y1, dinv, b1[None, :], W2)

    s2 = jax.ops.segment_sum(y2[src], dst, num_segments=N)

    pooled_sum = pl.pallas_call(
        _pool_kernel,
        grid=grid,
        in_specs=[row_spec(HID), row_spec(HID), dinv_spec,
                  pl.BlockSpec((1, HID), lambda i: (0, 0))],
        out_specs=pl.BlockSpec((1, HID), lambda i: (0, 0)),
        out_shape=jax.ShapeDtypeStruct((1, HID), jnp.float32),
    )(s2, y2, dinv, b2[None, :])

    return pl.pallas_call(
        _head_kernel,
        out_shape=jax.ShapeDtypeStruct((1, 2), jnp.float32),
    )(pooled_sum, climber_features[None, :], Wg, bg[None, :],
      Wc1, bc1[None, :], Wc2, bc2[None, :])


# speedup vs baseline: 3.9271x; 1.4546x over previous
import functools

import jax
import jax.numpy as jnp
from jax import lax
from jax.experimental import pallas as pl
from jax.experimental.pallas import tpu as pltpu
from jax.experimental.pallas import tpu_sc as plsc

N = 10000
E = 320000
D_IN = 128
HID = 256
RB = 1000
NC, NS, L = 2, 16, 16
CHUNK = 128
CHUNKS = 2560
E_PAD = CHUNKS * CHUNK
NCH = CHUNKS // NS
DCH = CHUNKS // (NS * NC)
N_ACC = 10240
RPT = N_ACC // NS
NSPL = N_ACC // 2
DUMP = NSPL
ACC_R = 5248
APT = ACC_R // NS
WPT = NSPL // NS

_mesh = plsc.VectorSubcoreMesh(core_axis_name="c", subcore_axis_name="s")


@functools.partial(
    pl.kernel, mesh=_mesh,
    out_type=jax.ShapeDtypeStruct((N_ACC, 128), jnp.float32),
    scratch_types=[
        pltpu.VMEM((NCH, CHUNK), jnp.int32),
        pltpu.VMEM((CHUNK, 128), jnp.float32),
        pltpu.VMEM_SHARED((ACC_R, 128), jnp.float32),
    ],
)
def _deg_sc(dstp, zrows, out, dstv, ones_v, acc):
    c = lax.axis_index("c")
    s = lax.axis_index("s")
    pltpu.sync_copy(dstp.at[c, pl.ds(s * NCH, NCH)], dstv)

    one16 = jnp.full((L,), 1.0, jnp.float32)

    def obody(j, _):
        def ibody(k, _):
            ones_v[j, pl.ds(k * L, L)] = one16
            return 0
        lax.fori_loop(0, 128 // L, ibody, 0)
        return 0

    lax.fori_loop(0, CHUNK, obody, 0)
    pltpu.sync_copy(zrows, acc.at[pl.ds(s * APT, APT)])
    plsc.subcore_barrier()

    def ebody(j, _):
        pltpu.sync_copy(ones_v, acc.at[dstv.at[j]], add=True)
        return 0

    lax.fori_loop(0, NCH, ebody, 0)
    plsc.subcore_barrier()
    pltpu.sync_copy(acc.at[pl.ds(s * WPT, WPT)],
                    out.at[pl.ds(c * NSPL + s * WPT, WPT)])


@functools.partial(
    pl.kernel, mesh=_mesh,
    out_type=[jax.ShapeDtypeStruct((N_ACC, 128), jnp.float32)] * 2,
    scratch_types=[
        pltpu.VMEM((NCH, CHUNK), jnp.int32),
        pltpu.VMEM((NCH, CHUNK), jnp.int32),
        pltpu.VMEM((CHUNK, 128), jnp.float32),
        pltpu.VMEM((CHUNK, 128), jnp.float32),
        pltpu.VMEM_SHARED((ACC_R, 128), jnp.float32),
        pltpu.SemaphoreType.DMA,
        pltpu.SemaphoreType.DMA,
    ],
)
def _seg_sc(yl, yh, src2, dstp, zrows, olo, ohi,
            srcv, dstv, ra, rb, acc, sa, sb):
    c = lax.axis_index("c")
    s = lax.axis_index("s")
    pltpu.sync_copy(src2.at[pl.ds(s * NCH, NCH)], srcv)

    def one_pass(y_hbm, o_hbm, k):
        pltpu.sync_copy(dstp.at[k, pl.ds(s * NCH, NCH)], dstv)
        pltpu.sync_copy(zrows, acc.at[pl.ds(s * APT, APT)])
        plsc.subcore_barrier()

        def body(p, _):
            j = p * 2
            ga = pltpu.async_copy(y_hbm.at[srcv.at[j]], ra, sa)
            gb = pltpu.async_copy(y_hbm.at[srcv.at[j + 1]], rb, sb)
            ga.wait()
            pltpu.sync_copy(ra, acc.at[dstv.at[j]], add=True)
            gb.wait()
            pltpu.sync_copy(rb, acc.at[dstv.at[j + 1]], add=True)
            return 0

        lax.fori_loop(0, NCH // 2, body, 0)
        plsc.subcore_barrier()
        pltpu.sync_copy(acc.at[pl.ds(s * WPT, WPT)],
                        o_hbm.at[pl.ds(k * NSPL + s * WPT, WPT)])
        plsc.subcore_barrier()

    @pl.when(c == 0)
    def _():
        one_pass(yl, olo, 0)
        one_pass(yl, olo, 1)

    @pl.when(c == 1)
    def _():
        one_pass(yh, ohi, 0)
        one_pass(yh, ohi, 1)


def _dinv_kernel(deg_ref, o_ref):
    d = deg_ref[:, 0:1] + 1.0
    o_ref[...] = lax.rsqrt(jnp.maximum(d, 1.0))


def _scale_mm_kernel(dinv_ref, x_ref, w_ref, olo_ref, ohi_ref):
    xw = jnp.dot(x_ref[...], w_ref[...], preferred_element_type=jnp.float32)
    y = dinv_ref[...] * xw
    olo_ref[...] = y[:, :128]
    ohi_ref[...] = y[:, 128:]


def _mid_kernel(slo, shi, ylo, yhi, dinv_ref, b_ref, w_ref, olo_ref, ohi_ref):
    sv = jnp.concatenate([slo[...], shi[...]], axis=1)
    yv = jnp.concatenate([ylo[...], yhi[...]], axis=1)
    h = jnp.maximum(dinv_ref[...] * (sv + yv) + b_ref[...], 0.0)
    o = dinv_ref[...] * jnp.dot(h, w_ref[...],
                                preferred_element_type=jnp.float32)
    olo_ref[...] = o[:, :128]
    ohi_ref[...] = o[:, 128:]


def _pool_kernel(slo, shi, ylo, yhi, dinv_ref, b_ref, acc_ref):
    @pl.when(pl.program_id(0) == 0)
    def _():
        acc_ref[...] = jnp.zeros_like(acc_ref)

    sv = jnp.concatenate([slo[...], shi[...]], axis=1)
    yv = jnp.concatenate([ylo[...], yhi[...]], axis=1)
    h = jnp.maximum(dinv_ref[...] * (sv + yv) + b_ref[...], 0.0)
    acc_ref[...] += jnp.sum(h, axis=0, keepdims=True)


def _head_kernel(ps_ref, cf_ref, wg_ref, bg_ref, wc1_ref, bc1_ref,
                 wc2_ref, bc2_ref, o_ref):
    pooled = ps_ref[...] * (1.0 / N)
    g = jnp.maximum(jnp.dot(cf_ref[...], wg_ref[...],
                            preferred_element_type=jnp.float32)
                    + bg_ref[...], 0.0)
    out = jnp.concatenate([pooled, g], axis=1)
    h = jnp.maximum(jnp.dot(out, wc1_ref[...],
                            preferred_element_type=jnp.float32)
                    + bc1_ref[...], 0.0)
    o_ref[...] = jnp.dot(h, wc2_ref[...],
                         preferred_element_type=jnp.float32) + bc2_ref[...]


def kernel(x, edge_index, climber_features, W1, b1, W2, b2, Wg, bg,
           Wc1, bc1, Wc2, bc2):
    src = edge_index[0].astype(jnp.int32)
    dst = edge_index[1].astype(jnp.int32)

    pad = E_PAD - E
    src2 = jnp.concatenate([src, jnp.zeros((pad,), jnp.int32)]
                           ).reshape(CHUNKS, CHUNK)
    dstf = jnp.concatenate([dst, jnp.full((pad,), N, jnp.int32)])
    d0 = jnp.where(dstf < NSPL, dstf, DUMP)
    d1r = dstf - NSPL
    d1 = jnp.where(dstf >= NSPL, d1r, DUMP)
    dstp = jnp.stack([d0, d1]).reshape(2, CHUNKS, CHUNK)
    zrows = jnp.zeros((APT, 128), jnp.float32)

    deg = _deg_sc(dstp, zrows)
    dinv = pl.pallas_call(
        _dinv_kernel,
        out_shape=jax.ShapeDtypeStruct((N_ACC, 1), jnp.float32),
    )(deg)

    grid = (N // RB,)
    row_spec = lambda d: pl.BlockSpec((RB, d), lambda i: (i, 0))
    dinv_spec = pl.BlockSpec((RB, 1), lambda i: (i, 0))
    half_out = [jax.ShapeDtypeStruct((N, 128), jnp.float32)] * 2
    half_specs = [row_spec(128)] * 2

    y1l, y1h = pl.pallas_call(
        _scale_mm_kernel,
        grid=grid,
        in_specs=[dinv_spec, row_spec(D_IN),
                  pl.BlockSpec((D_IN, HID), lambda i: (0, 0))],
        out_specs=half_specs,
        out_shape=half_out,
    )(dinv, x, W1)

    s1l, s1h = _seg_sc(y1l, y1h, src2, dstp, zrows)

    y2l, y2h = pl.pallas_call(
        _mid_kernel,
        grid=grid,
        in_specs=half_specs + half_specs +
                 [dinv_spec, pl.BlockSpec((1, HID), lambda i: (0, 0)),
                  pl.BlockSpec((HID, HID), lambda i: (0, 0))],
        out_specs=half_specs,
        out_shape=half_out,
    )(s1l, s1h, y1l, y1h, dinv, b1[None, :], W2)

    s2l, s2h = _seg_sc(y2l, y2h, src2, dstp, zrows)

    pooled_sum = pl.pallas_call(
        _pool_kernel,
        grid=grid,
        in_specs=half_specs + half_specs +
                 [dinv_spec, pl.BlockSpec((1, HID), lambda i: (0, 0))],
        out_specs=pl.BlockSpec((1, HID), lambda i: (0, 0)),
        out_shape=jax.ShapeDtypeStruct((1, HID), jnp.float32),
    )(s2l, s2h, y2l, y2h, dinv, b2[None, :])

    return pl.pallas_call(
        _head_kernel,
        out_shape=jax.ShapeDtypeStruct((1, 2), jnp.float32),
    )(pooled_sum, climber_features[None, :], Wg, bg[None, :],
      Wc1, bc1[None, :], Wc2, bc2[None, :])


# speedup vs baseline: 8.4876x; 2.1613x over previous
import functools

import jax
import jax.numpy as jnp
from jax import lax
from jax.experimental import pallas as pl
from jax.experimental.pallas import tpu as pltpu
from jax.experimental.pallas import tpu_sc as plsc

N = 10000
E = 320000
D_IN = 128
HID = 256
RB = 1000
NC, NS, L = 2, 16, 16
CH = 64
CHUNKS = 5120
E_PAD = CHUNKS * CH
TCH = CHUNKS // NS
PIECE = 32
NPIECE = TCH // PIECE
DTCH = CHUNKS // (NS * NC)
N_ACC = 10240
WPT = N_ACC // NS
NBUF = 4

_mesh = plsc.VectorSubcoreMesh(core_axis_name="c", subcore_axis_name="s")


@functools.partial(
    pl.kernel, mesh=_mesh,
    out_type=jax.ShapeDtypeStruct((NC, N_ACC, 128), jnp.float32),
    scratch_types=[
        pltpu.VMEM((DTCH, CH), jnp.int32),
        pltpu.VMEM((CH, 128), jnp.float32),
        pltpu.VMEM_SHARED((N_ACC, 128), jnp.float32),
    ],
)
def _deg_sc(dst2, zrows, out, dstv, ones_v, acc):
    c = lax.axis_index("c")
    s = lax.axis_index("s")
    pltpu.sync_copy(dst2.at[pl.ds((c * NS + s) * DTCH, DTCH)], dstv)

    one16 = jnp.full((L,), 1.0, jnp.float32)

    def obody(j, _):
        def ibody(k, _):
            ones_v[j, pl.ds(k * L, L)] = one16
            return 0
        lax.fori_loop(0, 128 // L, ibody, 0)
        return 0

    lax.fori_loop(0, CH, obody, 0)
    sl = pl.ds(s * WPT, WPT)
    pltpu.sync_copy(zrows, acc.at[sl])
    plsc.subcore_barrier()

    def ebody(j, _):
        pltpu.sync_copy(ones_v, acc.at[dstv.at[j]], add=True)
        return 0

    lax.fori_loop(0, DTCH, ebody, 0)
    plsc.subcore_barrier()
    pltpu.sync_copy(acc.at[sl], out.at[c, sl])


@functools.partial(
    pl.kernel, mesh=_mesh,
    out_type=[jax.ShapeDtypeStruct((N_ACC, 128), jnp.float32)] * 2,
    scratch_types=(
        [pltpu.VMEM((PIECE, CH), jnp.int32)] * 4 +
        [pltpu.VMEM((CH, 128), jnp.float32)] * NBUF +
        [pltpu.VMEM_SHARED((N_ACC, 128), jnp.float32)] +
        [pltpu.SemaphoreType.DMA] * (NBUF + 1)
    ),
)
def _seg_sc(yl, yh, src2, dst2, zrows, olo, ohi,
            srcA, dstA, srcB, dstB, *rest):
    rbuf = rest[:NBUF]
    acc = rest[NBUF]
    gsem = rest[NBUF + 1:2 * NBUF + 1]
    isem = rest[2 * NBUF + 1]
    cid = lax.axis_index("c")
    s = lax.axis_index("s")
    base = s * TCH
    sl = pl.ds(s * WPT, WPT)

    def run(y_hbm, o_hbm):
        pltpu.sync_copy(zrows, acc.at[sl])
        pltpu.sync_copy(src2.at[pl.ds(base, PIECE)], srcA)
        pltpu.sync_copy(dst2.at[pl.ds(base, PIECE)], dstA)
        plsc.subcore_barrier()
        for b in range(NBUF):
            pltpu.async_copy(y_hbm.at[srcA.at[b]], rbuf[b], gsem[b])

        def piece(p, cur_s, cur_d, nxt_s, nxt_d, has_next):
            if has_next:
                nb = base + (p + 1) * PIECE
                pltpu.async_copy(src2.at[pl.ds(nb, PIECE)], nxt_s, isem)
                pltpu.async_copy(dst2.at[pl.ds(nb, PIECE)], nxt_d, isem)
            for cc in range(PIECE):
                b = cc % NBUF
                pltpu.make_async_copy(y_hbm.at[cur_s.at[cc]], rbuf[b],
                                      gsem[b]).wait()
                pltpu.sync_copy(rbuf[b], acc.at[cur_d.at[cc]], add=True)
                if cc + NBUF < PIECE:
                    pltpu.async_copy(y_hbm.at[cur_s.at[cc + NBUF]], rbuf[b],
                                     gsem[b])
                elif has_next:
                    if cc == PIECE - NBUF:
                        nb2 = base + (p + 1) * PIECE
                        pltpu.make_async_copy(src2.at[pl.ds(nb2, PIECE)],
                                              nxt_s, isem).wait()
                        pltpu.make_async_copy(dst2.at[pl.ds(nb2, PIECE)],
                                              nxt_d, isem).wait()
                    pltpu.async_copy(y_hbm.at[nxt_s.at[cc + NBUF - PIECE]],
                                     rbuf[b], gsem[b])

        def body(r, _):
            p = r * 2
            piece(p, srcA, dstA, srcB, dstB, True)
            piece(p + 1, srcB, dstB, srcA, dstA, True)
            return 0

        lax.fori_loop(0, NPIECE // 2 - 1, body, 0)
        piece(NPIECE - 2, srcA, dstA, srcB, dstB, True)
        piece(NPIECE - 1, srcB, dstB, srcA, dstA, False)

        plsc.subcore_barrier()
        pltpu.sync_copy(acc.at[sl], o_hbm.at[sl])

    @pl.when(cid == 0)
    def _():
        run(yl, olo)

    @pl.when(cid == 1)
    def _():
        run(yh, ohi)


def _dinv_kernel(deg_ref, o_ref):
    d = deg_ref[0][:, 0:1] + deg_ref[1][:, 0:1] + 1.0
    o_ref[...] = lax.rsqrt(jnp.maximum(d, 1.0))


def _scale_mm_kernel(dinv_ref, x_ref, w_ref, olo_ref, ohi_ref):
    xw = jnp.dot(x_ref[...], w_ref[...], preferred_element_type=jnp.float32)
    y = dinv_ref[...] * xw
    olo_ref[...] = y[:, :128]
    ohi_ref[...] = y[:, 128:]


def _mid_kernel(slo, shi, ylo, yhi, dinv_ref, b_ref, w_ref, olo_ref, ohi_ref):
    sv = jnp.concatenate([slo[...], shi[...]], axis=1)
    yv = jnp.concatenate([ylo[...], yhi[...]], axis=1)
    h = jnp.maximum(dinv_ref[...] * (sv + yv) + b_ref[...], 0.0)
    o = dinv_ref[...] * jnp.dot(h, w_ref[...],
                                preferred_element_type=jnp.float32)
    olo_ref[...] = o[:, :128]
    ohi_ref[...] = o[:, 128:]


def _pool_kernel(slo, shi, ylo, yhi, dinv_ref, b_ref, acc_ref):
    @pl.when(pl.program_id(0) == 0)
    def _():
        acc_ref[...] = jnp.zeros_like(acc_ref)

    sv = jnp.concatenate([slo[...], shi[...]], axis=1)
    yv = jnp.concatenate([ylo[...], yhi[...]], axis=1)
    h = jnp.maximum(dinv_ref[...] * (sv + yv) + b_ref[...], 0.0)
    acc_ref[...] += jnp.sum(h, axis=0, keepdims=True)


def _head_kernel(ps_ref, cf_ref, wg_ref, bg_ref, wc1_ref, bc1_ref,
                 wc2_ref, bc2_ref, o_ref):
    pooled = ps_ref[...] * (1.0 / N)
    g = jnp.maximum(jnp.dot(cf_ref[...], wg_ref[...],
                            preferred_element_type=jnp.float32)
                    + bg_ref[...], 0.0)
    out = jnp.concatenate([pooled, g], axis=1)
    h = jnp.maximum(jnp.dot(out, wc1_ref[...],
                            preferred_element_type=jnp.float32)
                    + bc1_ref[...], 0.0)
    o_ref[...] = jnp.dot(h, wc2_ref[...],
                         preferred_element_type=jnp.float32) + bc2_ref[...]


def kernel(x, edge_index, climber_features, W1, b1, W2, b2, Wg, bg,
           Wc1, bc1, Wc2, bc2):
    src = edge_index[0].astype(jnp.int32)
    dst = edge_index[1].astype(jnp.int32)

    pad = E_PAD - E
    src2 = jnp.concatenate([src, jnp.zeros((pad,), jnp.int32)]
                           ).reshape(CHUNKS, CH)
    dst2 = jnp.concatenate([dst, jnp.full((pad,), N, jnp.int32)]
                           ).reshape(CHUNKS, CH)
    zrows = jnp.zeros((WPT, 128), jnp.float32)

    deg = _deg_sc(dst2, zrows)
    dinv = pl.pallas_call(
        _dinv_kernel,
        out_shape=jax.ShapeDtypeStruct((N_ACC, 1), jnp.float32),
    )(deg)

    grid = (N // RB,)
    row_spec = lambda d: pl.BlockSpec((RB, d), lambda i: (i, 0))
    dinv_spec = pl.BlockSpec((RB, 1), lambda i: (i, 0))
    half_out = [jax.ShapeDtypeStruct((N, 128), jnp.float32)] * 2
    half_specs = [row_spec(128)] * 2

    y1l, y1h = pl.pallas_call(
        _scale_mm_kernel,
        grid=grid,
        in_specs=[dinv_spec, row_spec(D_IN),
                  pl.BlockSpec((D_IN, HID), lambda i: (0, 0))],
        out_specs=half_specs,
        out_shape=half_out,
    )(dinv, x, W1)

    s1l, s1h = _seg_sc(y1l, y1h, src2, dst2, zrows)

    y2l, y2h = pl.pallas_call(
        _mid_kernel,
        grid=grid,
        in_specs=half_specs + half_specs +
                 [dinv_spec, pl.BlockSpec((1, HID), lambda i: (0, 0)),
                  pl.BlockSpec((HID, HID), lambda i: (0, 0))],
        out_specs=half_specs,
        out_shape=half_out,
    )(s1l, s1h, y1l, y1h, dinv, b1[None, :], W2)

    s2l, s2h = _seg_sc(y2l, y2h, src2, dst2, zrows)

    pooled_sum = pl.pallas_call(
        _pool_kernel,
        grid=grid,
        in_specs=half_specs + half_specs +
                 [dinv_spec, pl.BlockSpec((1, HID), lambda i: (0, 0))],
        out_specs=pl.BlockSpec((1, HID), lambda i: (0, 0)),
        out_shape=jax.ShapeDtypeStruct((1, HID), jnp.float32),
    )(s2l, s2h, y2l, y2h, dinv, b2[None, :])

    return pl.pallas_call(
        _head_kernel,
        out_shape=jax.ShapeDtypeStruct((1, 2), jnp.float32),
    )(pooled_sum, climber_features[None, :], Wg, bg[None, :],
      Wc1, bc1[None, :], Wc2, bc2[None, :])


# speedup vs baseline: 9.0092x; 1.0615x over previous
import functools

import jax
import jax.numpy as jnp
from jax import lax
from jax.experimental import pallas as pl
from jax.experimental.pallas import tpu as pltpu
from jax.experimental.pallas import tpu_sc as plsc

N = 10000
E = 320000
D_IN = 128
HID = 256
RB = 1000
NC, NS, L = 2, 16, 16
CH = 32
CHUNKS = 10240
E_PAD = CHUNKS * CH
TCH = CHUNKS // NS
PIECE = 32
NPIECE = TCH // PIECE
DTCH = CHUNKS // (NS * NC)
N_ACC = 10240
WPT = N_ACC // NS
NBUF = 8

_mesh = plsc.VectorSubcoreMesh(core_axis_name="c", subcore_axis_name="s")


@functools.partial(
    pl.kernel, mesh=_mesh,
    out_type=jax.ShapeDtypeStruct((NC, N_ACC, 128), jnp.float32),
    scratch_types=[
        pltpu.VMEM((DTCH, CH), jnp.int32),
        pltpu.VMEM((CH, 128), jnp.float32),
        pltpu.VMEM_SHARED((N_ACC, 128), jnp.float32),
    ],
)
def _deg_sc(dst2, zrows, out, dstv, ones_v, acc):
    c = lax.axis_index("c")
    s = lax.axis_index("s")
    pltpu.sync_copy(dst2.at[pl.ds((c * NS + s) * DTCH, DTCH)], dstv)

    one16 = jnp.full((L,), 1.0, jnp.float32)

    def obody(j, _):
        def ibody(k, _):
            ones_v[j, pl.ds(k * L, L)] = one16
            return 0
        lax.fori_loop(0, 128 // L, ibody, 0)
        return 0

    lax.fori_loop(0, CH, obody, 0)
    sl = pl.ds(s * WPT, WPT)
    pltpu.sync_copy(zrows, acc.at[sl])
    plsc.subcore_barrier()

    def ebody(j, _):
        pltpu.sync_copy(ones_v, acc.at[dstv.at[j]], add=True)
        return 0

    lax.fori_loop(0, DTCH, ebody, 0)
    plsc.subcore_barrier()
    pltpu.sync_copy(acc.at[sl], out.at[c, sl])


@functools.partial(
    pl.kernel, mesh=_mesh,
    out_type=[jax.ShapeDtypeStruct((N_ACC, 128), jnp.float32)] * 2,
    scratch_types=(
        [pltpu.VMEM((PIECE, CH), jnp.int32)] * 4 +
        [pltpu.VMEM((CH, 128), jnp.float32)] * NBUF +
        [pltpu.VMEM_SHARED((N_ACC, 128), jnp.float32)] +
        [pltpu.SemaphoreType.DMA] * (NBUF + 1)
    ),
)
def _seg_sc(yl, yh, src2, dst2, zrows, olo, ohi,
            srcA, dstA, srcB, dstB, *rest):
    rbuf = rest[:NBUF]
    acc = rest[NBUF]
    gsem = rest[NBUF + 1:2 * NBUF + 1]
    isem = rest[2 * NBUF + 1]
    cid = lax.axis_index("c")
    s = lax.axis_index("s")
    base = s * TCH
    sl = pl.ds(s * WPT, WPT)

    def run(y_hbm, o_hbm):
        pltpu.sync_copy(zrows, acc.at[sl])
        pltpu.sync_copy(src2.at[pl.ds(base, PIECE)], srcA)
        pltpu.sync_copy(dst2.at[pl.ds(base, PIECE)], dstA)
        plsc.subcore_barrier()
        for b in range(NBUF):
            pltpu.async_copy(y_hbm.at[srcA.at[b]], rbuf[b], gsem[b])

        def piece(p, cur_s, cur_d, nxt_s, nxt_d, has_next):
            if has_next:
                nb = base + (p + 1) * PIECE
                pltpu.async_copy(src2.at[pl.ds(nb, PIECE)], nxt_s, isem)
                pltpu.async_copy(dst2.at[pl.ds(nb, PIECE)], nxt_d, isem)
            for cc in range(PIECE):
                b = cc % NBUF
                pltpu.make_async_copy(y_hbm.at[cur_s.at[cc]], rbuf[b],
                                      gsem[b]).wait()
                pltpu.sync_copy(rbuf[b], acc.at[cur_d.at[cc]], add=True)
                if cc + NBUF < PIECE:
                    pltpu.async_copy(y_hbm.at[cur_s.at[cc + NBUF]], rbuf[b],
                                     gsem[b])
                elif has_next:
                    if cc == PIECE - NBUF:
                        nb2 = base + (p + 1) * PIECE
                        pltpu.make_async_copy(src2.at[pl.ds(nb2, PIECE)],
                                              nxt_s, isem).wait()
                        pltpu.make_async_copy(dst2.at[pl.ds(nb2, PIECE)],
                                              nxt_d, isem).wait()
                    pltpu.async_copy(y_hbm.at[nxt_s.at[cc + NBUF - PIECE]],
                                     rbuf[b], gsem[b])

        def body(r, _):
            p = r * 2
            piece(p, srcA, dstA, srcB, dstB, True)
            piece(p + 1, srcB, dstB, srcA, dstA, True)
            return 0

        lax.fori_loop(0, NPIECE // 2 - 1, body, 0)
        piece(NPIECE - 2, srcA, dstA, srcB, dstB, True)
        piece(NPIECE - 1, srcB, dstB, srcA, dstA, False)

        plsc.subcore_barrier()
        pltpu.sync_copy(acc.at[sl], o_hbm.at[sl])

    @pl.when(cid == 0)
    def _():
        run(yl, olo)

    @pl.when(cid == 1)
    def _():
        run(yh, ohi)


def _dinv_kernel(deg_ref, o_ref):
    d = deg_ref[0][:, 0:1] + deg_ref[1][:, 0:1] + 1.0
    o_ref[...] = lax.rsqrt(jnp.maximum(d, 1.0))


def _scale_mm_kernel(dinv_ref, x_ref, w_ref, olo_ref, ohi_ref):
    xw = jnp.dot(x_ref[...], w_ref[...], preferred_element_type=jnp.float32)
    y = dinv_ref[...] * xw
    olo_ref[...] = y[:, :128]
    ohi_ref[...] = y[:, 128:]


def _mid_kernel(slo, shi, ylo, yhi, dinv_ref, b_ref, w_ref, olo_ref, ohi_ref):
    sv = jnp.concatenate([slo[...], shi[...]], axis=1)
    yv = jnp.concatenate([ylo[...], yhi[...]], axis=1)
    h = jnp.maximum(dinv_ref[...] * (sv + yv) + b_ref[...], 0.0)
    o = dinv_ref[...] * jnp.dot(h, w_ref[...],
                                preferred_element_type=jnp.float32)
    olo_ref[...] = o[:, :128]
    ohi_ref[...] = o[:, 128:]


def _pool_kernel(slo, shi, ylo, yhi, dinv_ref, b_ref, acc_ref):
    @pl.when(pl.program_id(0) == 0)
    def _():
        acc_ref[...] = jnp.zeros_like(acc_ref)

    sv = jnp.concatenate([slo[...], shi[...]], axis=1)
    yv = jnp.concatenate([ylo[...], yhi[...]], axis=1)
    h = jnp.maximum(dinv_ref[...] * (sv + yv) + b_ref[...], 0.0)
    acc_ref[...] += jnp.sum(h, axis=0, keepdims=True)


def _head_kernel(ps_ref, cf_ref, wg_ref, bg_ref, wc1_ref, bc1_ref,
                 wc2_ref, bc2_ref, o_ref):
    pooled = ps_ref[...] * (1.0 / N)
    g = jnp.maximum(jnp.dot(cf_ref[...], wg_ref[...],
                            preferred_element_type=jnp.float32)
                    + bg_ref[...], 0.0)
    out = jnp.concatenate([pooled, g], axis=1)
    h = jnp.maximum(jnp.dot(out, wc1_ref[...],
                            preferred_element_type=jnp.float32)
                    + bc1_ref[...], 0.0)
    o_ref[...] = jnp.dot(h, wc2_ref[...],
                         preferred_element_type=jnp.float32) + bc2_ref[...]


def kernel(x, edge_index, climber_features, W1, b1, W2, b2, Wg, bg,
           Wc1, bc1, Wc2, bc2):
    src = edge_index[0].astype(jnp.int32)
    dst = edge_index[1].astype(jnp.int32)

    pad = E_PAD - E
    src2 = jnp.concatenate([src, jnp.zeros((pad,), jnp.int32)]
                           ).reshape(CHUNKS, CH)
    dst2 = jnp.concatenate([dst, jnp.full((pad,), N, jnp.int32)]
                           ).reshape(CHUNKS, CH)
    zrows = jnp.zeros((WPT, 128), jnp.float32)

    deg = _deg_sc(dst2, zrows)
    dinv = pl.pallas_call(
        _dinv_kernel,
        out_shape=jax.ShapeDtypeStruct((N_ACC, 1), jnp.float32),
    )(deg)

    grid = (N // RB,)
    row_spec = lambda d: pl.BlockSpec((RB, d), lambda i: (i, 0))
    dinv_spec = pl.BlockSpec((RB, 1), lambda i: (i, 0))
    half_out = [jax.ShapeDtypeStruct((N, 128), jnp.float32)] * 2
    half_specs = [row_spec(128)] * 2

    y1l, y1h = pl.pallas_call(
        _scale_mm_kernel,
        grid=grid,
        in_specs=[dinv_spec, row_spec(D_IN),
                  pl.BlockSpec((D_IN, HID), lambda i: (0, 0))],
        out_specs=half_specs,
        out_shape=half_out,
    )(dinv, x, W1)

    s1l, s1h = _seg_sc(y1l, y1h, src2, dst2, zrows)

    y2l, y2h = pl.pallas_call(
        _mid_kernel,
        grid=grid,
        in_specs=half_specs + half_specs +
                 [dinv_spec, pl.BlockSpec((1, HID), lambda i: (0, 0)),
                  pl.BlockSpec((HID, HID), lambda i: (0, 0))],
        out_specs=half_specs,
        out_shape=half_out,
    )(s1l, s1h, y1l, y1h, dinv, b1[None, :], W2)

    s2l, s2h = _seg_sc(y2l, y2h, src2, dst2, zrows)

    pooled_sum = pl.pallas_call(
        _pool_kernel,
        grid=grid,
        in_specs=half_specs + half_specs +
                 [dinv_spec, pl.BlockSpec((1, HID), lambda i: (0, 0))],
        out_specs=pl.BlockSpec((1, HID), lambda i: (0, 0)),
        out_shape=jax.ShapeDtypeStruct((1, HID), jnp.float32),
    )(s2l, s2h, y2l, y2h, dinv, b2[None, :])

    return pl.pallas_call(
        _head_kernel,
        out_shape=jax.ShapeDtypeStruct((1, 2), jnp.float32),
    )(pooled_sum, climber_features[None, :], Wg, bg[None, :],
      Wc1, bc1[None, :], Wc2, bc2[None, :])
